# baseline (speedup 1.0000x reference)
import functools

import jax
import jax.numpy as jnp
import numpy as np
from jax import lax
from jax.experimental import pallas as pl
from jax.experimental.pallas import tpu as pltpu
from jax.experimental.pallas import tpu_sc as plsc

RATIO = 0.75


@functools.lru_cache(maxsize=None)
def _make_gather(num_rows, C, T, B, NC, NS, n_chunks, chunk, depth):
    NW = NC * NS
    seg = T * B // NW
    mesh = plsc.VectorSubcoreMesh(core_axis_name="c", subcore_axis_name="s")

    @functools.partial(
        pl.kernel,
        mesh=mesh,
        out_type=(
            jax.ShapeDtypeStruct((num_rows, C), jnp.float32),
            jax.ShapeDtypeStruct((T * B,), jnp.int32),
            jax.ShapeDtypeStruct((T * B,), jnp.int32),
        ),
        scratch_types=[pltpu.VMEM((n_chunks, chunk), jnp.int32)]
        + [pltpu.VMEM((chunk, C), jnp.float32) for _ in range(depth)]
        + [
            pltpu.VMEM((2 * seg,), jnp.int32),
            pltpu.SemaphoreType.DMA,
            pltpu.SemaphoreType.DMA,
            pltpu.SemaphoreType.DMA,
        ],
    )
    def gather_k(table_hbm, idx_hbm, fb_hbm, out_hbm, fwd_hbm, bwd_hbm, idx_v, *rest):
        bufs = rest[:depth]
        fb_v, gsem, ssem, fbsem = rest[depth:depth + 4]
        wid = lax.axis_index("s") * NC + lax.axis_index("c")
        fb_in = pltpu.async_copy(fb_hbm.at[pl.ds(wid * 2 * seg, 2 * seg)], fb_v, fbsem)
        pltpu.sync_copy(idx_hbm.at[wid], idx_v)
        base = wid * (n_chunks * chunk)
        gathers = [None] * n_chunks
        scatters = [None] * n_chunks
        for c in range(min(depth, n_chunks)):
            gathers[c] = pltpu.async_copy(
                table_hbm.at[idx_v.at[c]], bufs[c % depth], gsem
            )
        for c in range(n_chunks):
            gathers[c].wait()
            scatters[c] = pltpu.async_copy(
                bufs[c % depth], out_hbm.at[pl.ds(base + c * chunk, chunk)], ssem
            )
            nxt = c + depth
            if nxt < n_chunks:
                scatters[nxt - depth].wait()
                gathers[nxt] = pltpu.async_copy(
                    table_hbm.at[idx_v.at[nxt]], bufs[nxt % depth], gsem
                )
        fb_in.wait()
        pltpu.sync_copy(fb_v.at[pl.ds(0, seg)], fwd_hbm.at[pl.ds(wid * seg, seg)])
        pltpu.sync_copy(fb_v.at[pl.ds(seg, seg)], bwd_hbm.at[pl.ds(wid * seg, seg)])
        for c in range(max(0, n_chunks - depth), n_chunks):
            scatters[c].wait()

    return gather_k


@functools.lru_cache(maxsize=None)
def _perm_indexes(T, B):
    with jax.ensure_compile_time_eval():
        perm_key = jax.random.key(42)
        keys = jax.random.split(perm_key, B)
        fwd = jnp.stack([jax.random.permutation(k, T) for k in keys], axis=-1)
        bwd = jnp.argsort(fwd, axis=0)
        return np.asarray(fwd), np.asarray(bwd)


def kernel(patches):
    T, B, C = patches.shape
    remain_T = int(T * (1 - RATIO))
    fwd_np, bwd_np = _perm_indexes(T, B)

    src_np = fwd_np[:remain_T] * B + np.arange(B, dtype=np.int32)[None, :]
    num_rows = remain_T * B

    info = plsc.get_sparse_core_info()
    NC, NS = info.num_cores, info.num_subcores
    NW = NC * NS
    rows_per_w = num_rows // NW
    assert rows_per_w * NW == num_rows
    chunk = 48
    depth = 3
    n_chunks = rows_per_w // chunk
    assert n_chunks * chunk == rows_per_w

    idx3 = jnp.asarray(src_np.reshape(NW, n_chunks, chunk).astype(np.int32))
    seg = T * B // NW
    fb_np = np.stack(
        [fwd_np.reshape(NW, seg).astype(np.int32),
         bwd_np.reshape(NW, seg).astype(np.int32)], axis=1
    ).reshape(NW * 2 * seg)
    fb = jnp.asarray(fb_np)
    table = patches.reshape(T * B, C)
    out_flat, fwd_flat, bwd_flat = _make_gather(
        num_rows, C, T, B, NC, NS, n_chunks, chunk, depth
    )(table, idx3, fb)
    return (
        out_flat.reshape(remain_T, B, C),
        fwd_flat.reshape(T, B),
        bwd_flat.reshape(T, B),
    )


# speedup vs baseline: 1.0618x; 1.0618x over previous
import functools

import jax
import jax.numpy as jnp
import numpy as np
from jax import lax
from jax.experimental import pallas as pl
from jax.experimental.pallas import tpu as pltpu
from jax.experimental.pallas import tpu_sc as plsc

RATIO = 0.75


@functools.lru_cache(maxsize=None)
def _make_gather(num_rows, C, NC, NS, n_chunks, chunk, depth, idx_pad):
    NW = NC * NS
    mesh = plsc.VectorSubcoreMesh(core_axis_name="c", subcore_axis_name="s")

    @functools.partial(
        pl.kernel,
        mesh=mesh,
        out_type=jax.ShapeDtypeStruct((num_rows, C), jnp.float32),
        scratch_types=[pltpu.VMEM((idx_pad,), jnp.int32)]
        + [pltpu.VMEM((chunk, C), jnp.float32) for _ in range(depth)]
        + [pltpu.SemaphoreType.DMA, pltpu.SemaphoreType.DMA],
    )
    def gather_k(table_hbm, idx_hbm, out_hbm, idx_v, *rest):
        bufs = rest[:depth]
        gsem, ssem = rest[depth], rest[depth + 1]
        wid = lax.axis_index("s") * NC + lax.axis_index("c")
        pltpu.sync_copy(idx_hbm.at[wid], idx_v)
        base = wid * (n_chunks * chunk)
        gathers = [None] * n_chunks
        scatters = [None] * n_chunks
        for c in range(min(depth, n_chunks)):
            gathers[c] = pltpu.async_copy(
                table_hbm.at[idx_v.at[pl.ds(c * chunk, chunk)]], bufs[c % depth], gsem
            )
        for c in range(n_chunks):
            gathers[c].wait()
            scatters[c] = pltpu.async_copy(
                bufs[c % depth], out_hbm.at[pl.ds(base + c * chunk, chunk)], ssem
            )
            nxt = c + depth
            if nxt < n_chunks:
                scatters[nxt - depth].wait()
                gathers[nxt] = pltpu.async_copy(
                    table_hbm.at[idx_v.at[pl.ds(nxt * chunk, chunk)]], bufs[nxt % depth], gsem
                )
        for c in range(max(0, n_chunks - depth), n_chunks):
            scatters[c].wait()

    return gather_k


@functools.lru_cache(maxsize=None)
def _perm_indexes(T, B):
    with jax.ensure_compile_time_eval():
        perm_key = jax.random.key(42)
        keys = jax.random.split(perm_key, B)
        fwd = jnp.stack([jax.random.permutation(k, T) for k in keys], axis=-1)
        bwd = jnp.argsort(fwd, axis=0)
        return np.asarray(fwd), np.asarray(bwd)


def kernel(patches):
    T, B, C = patches.shape
    remain_T = int(T * (1 - RATIO))
    fwd_np, bwd_np = _perm_indexes(T, B)

    src_np = fwd_np[:remain_T] * B + np.arange(B, dtype=np.int32)[None, :]
    num_rows = remain_T * B

    info = plsc.get_sparse_core_info()
    NC, NS = info.num_cores, info.num_subcores
    NW = NC * NS
    rows_per_w = num_rows // NW
    assert rows_per_w * NW == num_rows
    chunk = 48
    depth = 3
    n_chunks = rows_per_w // chunk
    assert n_chunks * chunk == rows_per_w

    idx_pad = -(-rows_per_w // 128) * 128
    idx_np = np.zeros((NW, idx_pad), dtype=np.int32)
    idx_np[:, :rows_per_w] = src_np.reshape(NW, rows_per_w)
    idx2 = jnp.asarray(idx_np)
    table = patches.reshape(T * B, C)
    out_flat = _make_gather(
        num_rows, C, NC, NS, n_chunks, chunk, depth, idx_pad
    )(table, idx2)
    return (
        out_flat.reshape(remain_T, B, C),
        jnp.asarray(fwd_np),
        jnp.asarray(bwd_np),
    )


# speedup vs baseline: 1.0937x; 1.0300x over previous
import functools

import jax
import jax.numpy as jnp
import numpy as np
from jax import lax
from jax.experimental import pallas as pl
from jax.experimental.pallas import tpu as pltpu
from jax.experimental.pallas import tpu_sc as plsc

RATIO = 0.75


@functools.lru_cache(maxsize=None)
def _make_gather(num_rows, C, NC, NS, n_chunks, chunk, depth, idx_pad):
    NW = NC * NS
    mesh = plsc.VectorSubcoreMesh(core_axis_name="c", subcore_axis_name="s")

    @functools.partial(
        pl.kernel,
        mesh=mesh,
        out_type=jax.ShapeDtypeStruct((num_rows, C), jnp.float32),
        scratch_types=[pltpu.VMEM((idx_pad,), jnp.int32)]
        + [pltpu.VMEM((chunk, C), jnp.float32) for _ in range(depth)]
        + [pltpu.SemaphoreType.DMA, pltpu.SemaphoreType.DMA],
    )
    def gather_k(table_hbm, idx_hbm, out_hbm, idx_v, *rest):
        bufs = rest[:depth]
        gsem, ssem = rest[depth], rest[depth + 1]
        wid = lax.axis_index("s") * NC + lax.axis_index("c")
        pltpu.sync_copy(idx_hbm.at[wid], idx_v)
        base = wid * (n_chunks * chunk)
        gathers = [None] * n_chunks
        scatters = [None] * n_chunks
        for c in range(min(depth, n_chunks)):
            gathers[c] = pltpu.async_copy(
                table_hbm.at[idx_v.at[pl.ds(c * chunk, chunk)]], bufs[c % depth], gsem
            )
        for c in range(n_chunks):
            gathers[c].wait()
            scatters[c] = pltpu.async_copy(
                bufs[c % depth], out_hbm.at[pl.ds(base + c * chunk, chunk)], ssem
            )
            nxt = c + depth
            if nxt < n_chunks:
                scatters[nxt - depth].wait()
                gathers[nxt] = pltpu.async_copy(
                    table_hbm.at[idx_v.at[pl.ds(nxt * chunk, chunk)]], bufs[nxt % depth], gsem
                )
        for c in range(max(0, n_chunks - depth), n_chunks):
            scatters[c].wait()

    return gather_k


@functools.lru_cache(maxsize=None)
def _make_split_fb(T, B):
    def split_k(fb_ref, fwd_ref, bwd_ref):
        fwd_ref[...] = fb_ref[:, :B]
        bwd_ref[...] = fb_ref[:, B:]

    return pl.pallas_call(
        split_k,
        out_shape=(
            jax.ShapeDtypeStruct((T, B), jnp.int32),
            jax.ShapeDtypeStruct((T, B), jnp.int32),
        ),
    )


@functools.lru_cache(maxsize=None)
def _perm_indexes(T, B):
    with jax.ensure_compile_time_eval():
        perm_key = jax.random.key(42)
        keys = jax.random.split(perm_key, B)
        fwd = jnp.stack([jax.random.permutation(k, T) for k in keys], axis=-1)
        bwd = jnp.argsort(fwd, axis=0)
        return np.asarray(fwd), np.asarray(bwd)


def kernel(patches):
    T, B, C = patches.shape
    remain_T = int(T * (1 - RATIO))
    fwd_np, bwd_np = _perm_indexes(T, B)

    src_np = fwd_np[:remain_T] * B + np.arange(B, dtype=np.int32)[None, :]
    num_rows = remain_T * B

    info = plsc.get_sparse_core_info()
    NC, NS = info.num_cores, info.num_subcores
    NW = NC * NS
    rows_per_w = num_rows // NW
    assert rows_per_w * NW == num_rows
    chunk = 48
    depth = 3
    n_chunks = rows_per_w // chunk
    assert n_chunks * chunk == rows_per_w

    idx_pad = -(-rows_per_w // 128) * 128
    idx_np = np.zeros((NW, idx_pad), dtype=np.int32)
    idx_np[:, :rows_per_w] = src_np.reshape(NW, rows_per_w)
    idx2 = jnp.asarray(idx_np)
    table = patches.reshape(T * B, C)
    out_flat = _make_gather(
        num_rows, C, NC, NS, n_chunks, chunk, depth, idx_pad
    )(table, idx2)
    fb = jnp.asarray(np.concatenate([fwd_np, bwd_np], axis=1).astype(np.int32))
    fwd, bwd = _make_split_fb(T, B)(fb)
    return out_flat.reshape(remain_T, B, C), fwd, bwd
